# baseline (speedup 1.0000x reference)
import functools

import jax
import jax.numpy as jnp
from jax import lax
from jax.experimental import pallas as pl
from jax.experimental.pallas import tpu as pltpu
from jax.experimental.pallas import tpu_sc as plsc

DIM = 1024
N_EXPERTS = 8
N_TOKENS = 2048
HID = 4 * DIM
HBLK = 512
N_HBLK = HID // HBLK

B = 256
NPAIR = 2 * N_TOKENS
PADTOT = NPAIR + N_EXPERTS * B
NT = PADTOT // B

NSUB = 32
TOK_PER_SUB = N_TOKENS // NSUB
ROWS_PER_SUB = PADTOT // NSUB

_MESH = dict(mesh=plsc.VectorSubcoreMesh(core_axis_name="c",
                                         subcore_axis_name="s"),
             compiler_params=pltpu.CompilerParams(needs_layout_passes=False))


def _wid():
    return lax.axis_index("s") * 2 + lax.axis_index("c")


def _iota16():
    return lax.broadcasted_iota(jnp.int32, (16,), 0)


def _k1_body(x_ref, wr_ref, br_ref, out_ref):
    out_ref[...] = (jnp.dot(x_ref[...], wr_ref[...],
                            preferred_element_type=jnp.float32)
                    + br_ref[...])


def _k1(x, Wr, br2):
    return pl.pallas_call(
        _k1_body,
        in_specs=[
            pl.BlockSpec((N_TOKENS, DIM), lambda: (0, 0)),
            pl.BlockSpec((DIM, N_EXPERTS), lambda: (0, 0)),
            pl.BlockSpec((1, N_EXPERTS), lambda: (0, 0)),
        ],
        out_specs=pl.BlockSpec((N_TOKENS, N_EXPERTS), lambda: (0, 0)),
        out_shape=jax.ShapeDtypeStruct((N_TOKENS, N_EXPERTS), jnp.float32),
    )(x, Wr, br2)


@functools.partial(
    pl.kernel,
    out_type=(
        jax.ShapeDtypeStruct((N_TOKENS,), jnp.int32),
        jax.ShapeDtypeStruct((N_TOKENS,), jnp.int32),
        jax.ShapeDtypeStruct((N_TOKENS,), jnp.float32),
        jax.ShapeDtypeStruct((N_TOKENS,), jnp.float32),
        jax.ShapeDtypeStruct((NSUB * 8,), jnp.int32),
    ),
    scratch_types=[
        pltpu.VMEM((TOK_PER_SUB * N_EXPERTS,), jnp.float32),
        pltpu.VMEM((TOK_PER_SUB,), jnp.int32),
        pltpu.VMEM((TOK_PER_SUB,), jnp.int32),
        pltpu.VMEM((TOK_PER_SUB,), jnp.float32),
        pltpu.VMEM((TOK_PER_SUB,), jnp.float32),
        pltpu.VMEM((16,), jnp.int32),
    ],
    **_MESH,
)
def _k2a(logits_hbm, e1_hbm, e2_hbm, w1_hbm, w2_hbm, histb_hbm,
         lg_v, e1_v, e2_v, w1_v, w2_v, hist_v):
    wid = _wid()
    base = wid * TOK_PER_SUB
    pltpu.sync_copy(
        logits_hbm.at[pl.ds(base * N_EXPERTS, TOK_PER_SUB * N_EXPERTS)],
        lg_v)
    it = _iota16()
    idx_vecs = []
    for c in range(TOK_PER_SUB // 16):
        rows = (it + c * 16) * N_EXPERTS
        L = [plsc.load_gather(lg_v, [rows + e]) for e in range(N_EXPERTS)]
        m1 = L[0]
        for e in range(1, N_EXPERTS):
            m1 = jnp.maximum(m1, L[e])
        idx1 = jnp.full((16,), N_EXPERTS - 1, jnp.int32)
        for e in range(N_EXPERTS - 1, -1, -1):
            idx1 = jnp.where(L[e] == m1, e, idx1)
        L2 = [jnp.where(idx1 == e, -3.0e38, L[e]) for e in range(N_EXPERTS)]
        m2 = L2[0]
        for e in range(1, N_EXPERTS):
            m2 = jnp.maximum(m2, L2[e])
        idx2 = jnp.full((16,), N_EXPERTS - 1, jnp.int32)
        for e in range(N_EXPERTS - 1, -1, -1):
            idx2 = jnp.where(L2[e] == m2, e, idx2)
        t = jnp.exp(m2 - m1)
        s = 1.0 + t
        e1_v[pl.ds(c * 16, 16)] = idx1
        e2_v[pl.ds(c * 16, 16)] = idx2
        w1_v[pl.ds(c * 16, 16)] = 1.0 / s
        w2_v[pl.ds(c * 16, 16)] = t / s
        idx_vecs += [idx1, idx2]
    hvec = jnp.zeros((16,), jnp.int32)
    for e in range(N_EXPERTS):
        tot = jnp.int32(0)
        for v in idx_vecs:
            tot = tot + lax.reduce_sum_p.bind(
                (v == e).astype(jnp.int32), axes=(0,))
        hvec = jnp.where(it == e, tot, hvec)
    hist_v[...] = hvec
    pltpu.sync_copy(e1_v, e1_hbm.at[pl.ds(base, TOK_PER_SUB)])
    pltpu.sync_copy(e2_v, e2_hbm.at[pl.ds(base, TOK_PER_SUB)])
    pltpu.sync_copy(w1_v, w1_hbm.at[pl.ds(base, TOK_PER_SUB)])
    pltpu.sync_copy(w2_v, w2_hbm.at[pl.ds(base, TOK_PER_SUB)])
    pltpu.sync_copy(hist_v.at[pl.ds(0, 8)], histb_hbm.at[pl.ds(wid * 8, 8)])


@functools.partial(
    pl.kernel,
    out_type=(
        jax.ShapeDtypeStruct((NPAIR,), jnp.int32),
        jax.ShapeDtypeStruct((PADTOT,), jnp.float32),
        jax.ShapeDtypeStruct((PADTOT, DIM), jnp.float32),
        jax.ShapeDtypeStruct((32,), jnp.int32),
        jax.ShapeDtypeStruct((32,), jnp.int32),
    ),
    scratch_types=[
        pltpu.VMEM((NSUB * 8,), jnp.int32),
        pltpu.VMEM((TOK_PER_SUB,), jnp.int32),
        pltpu.VMEM((TOK_PER_SUB,), jnp.int32),
        pltpu.VMEM((TOK_PER_SUB,), jnp.float32),
        pltpu.VMEM((TOK_PER_SUB,), jnp.float32),
        pltpu.VMEM((TOK_PER_SUB, DIM), jnp.float32),
        pltpu.VMEM((TOK_PER_SUB,), jnp.int32),
        pltpu.VMEM((TOK_PER_SUB,), jnp.int32),
        pltpu.VMEM((128,), jnp.int32),
        pltpu.VMEM((128,), jnp.float32),
        pltpu.VMEM((16,), jnp.int32),
        pltpu.VMEM((16,), jnp.int32),
        pltpu.SemaphoreType.DMA,
        pltpu.SemaphoreType.DMA,
    ],
    **_MESH,
)
def _k2b(e1_hbm, e2_hbm, w1_hbm, w2_hbm, histb_hbm, x_hbm,
         inv_hbm, sw_hbm, xs_hbm, te_hbm, ta_hbm,
         histall_v, e1_v, e2_v, w1_v, w2_v, xrows_v, pos1_v, pos2_v,
         posall_v, wall_v, tstage_v, astage_v, sem_in, sem_out):
    wid = _wid()
    base = wid * TOK_PER_SUB
    loads = [
        pltpu.async_copy(x_hbm.at[pl.ds(base, TOK_PER_SUB), :], xrows_v,
                         sem_in),
        pltpu.async_copy(histb_hbm, histall_v, sem_in),
        pltpu.async_copy(e1_hbm.at[pl.ds(base, TOK_PER_SUB)], e1_v, sem_in),
        pltpu.async_copy(e2_hbm.at[pl.ds(base, TOK_PER_SUB)], e2_v, sem_in),
        pltpu.async_copy(w1_hbm.at[pl.ds(base, TOK_PER_SUB)], w1_v, sem_in),
        pltpu.async_copy(w2_hbm.at[pl.ds(base, TOK_PER_SUB)], w2_v, sem_in),
    ]
    for h in loads:
        h.wait()
    it = _iota16()

    def rsum(v):
        return lax.reduce_sum_p.bind(v, axes=(0,))

    tot = []
    pre = []
    for e in range(N_EXPERTS):
        clo = plsc.load_gather(histall_v, [it * 8 + e])
        chi = plsc.load_gather(histall_v, [(it + 16) * 8 + e])
        tot.append(rsum(clo) + rsum(chi))
        z = jnp.zeros((16,), jnp.int32)
        pre.append(rsum(jnp.where(it < wid, clo, z))
                   + rsum(jnp.where(it + 16 < wid, chi, z)))
    off = [jnp.int32(0)]
    for e in range(N_EXPERTS):
        off.append(off[e] + ((tot[e] + (B - 1)) // B) * B)
    start = [off[e] + pre[e] for e in range(N_EXPERTS)]

    run = {e: jnp.int32(0) for e in range(N_EXPERTS)}
    for slot, (vsrc, wsrc) in enumerate(((e1_v, w1_v), (e2_v, w2_v))):
        for c in range(TOK_PER_SUB // 16):
            v = vsrc[pl.ds(c * 16, 16)]
            pos = jnp.zeros((16,), jnp.int32)
            for e in range(N_EXPERTS):
                m = v == e
                mi = m.astype(jnp.int32)
                r = plsc.cumsum(mi) - 1
                pos = jnp.where(m, start[e] + run[e] + r, pos)
                run[e] = run[e] + rsum(mi)
            o = slot * TOK_PER_SUB + c * 16
            posall_v[pl.ds(o, 16)] = pos
            wall_v[pl.ds(o, 16)] = wsrc[pl.ds(c * 16, 16)]
            pdst = pos1_v if slot == 0 else pos2_v
            pdst[pl.ds(c * 16, 16)] = pos
    stores = [
        pltpu.async_copy(pos1_v, inv_hbm.at[pl.ds(base, TOK_PER_SUB)],
                         sem_out),
        pltpu.async_copy(pos2_v,
                         inv_hbm.at[pl.ds(N_TOKENS + base, TOK_PER_SUB)],
                         sem_out),
        pltpu.async_copy(wall_v, sw_hbm.at[posall_v], sem_out),
        pltpu.async_copy(xrows_v, xs_hbm.at[pos1_v], sem_out),
        pltpu.async_copy(xrows_v, xs_hbm.at[pos2_v], sem_out),
    ]
    for h in stores:
        h.wait()

    @pl.when(wid == 0)
    def _tiles():
        for chunk in range(2):
            t_ids = it + chunk * 16
            tB = t_ids * B
            act = (tB < off[N_EXPERTS]).astype(jnp.int32)
            te = jnp.zeros((16,), jnp.int32)
            for k in range(1, N_EXPERTS):
                te = te + (tB >= off[k]).astype(jnp.int32)
            tstage_v[...] = te
            astage_v[...] = act
            pltpu.sync_copy(tstage_v, te_hbm.at[pl.ds(chunk * 16, 16)])
            pltpu.sync_copy(astage_v, ta_hbm.at[pl.ds(chunk * 16, 16)])


def _k4_body(te_ref, ta_ref, xs_ref, sw_ref, w1_ref, b1_ref, w2_ref, b2_ref,
             out_ref):
    t = pl.program_id(0)

    @pl.when(ta_ref[t] == 1)
    def _active():
        x = xs_ref[...].astype(jnp.bfloat16)
        acc = b2_ref[0] * 1.0
        for hb in range(N_HBLK):
            w1blk = w1_ref[0][:, hb * HBLK:(hb + 1) * HBLK]
            g = jnp.dot(x, w1blk, preferred_element_type=jnp.float32)
            g = g + b1_ref[0][:, hb * HBLK:(hb + 1) * HBLK]
            g = g * 0.5 * (1.0 + lax.erf(g * 0.7071067811865476))
            w2blk = w2_ref[0][hb * HBLK:(hb + 1) * HBLK, :]
            acc = acc + jnp.dot(g.astype(jnp.bfloat16), w2blk,
                                preferred_element_type=jnp.float32)
        out_ref[...] = acc * sw_ref[...]


def _k4(tile_e, tile_act, xs, sw2, W1, b1r, W2, b2):
    grid_spec = pltpu.PrefetchScalarGridSpec(
        num_scalar_prefetch=2,
        grid=(NT,),
        in_specs=[
            pl.BlockSpec((B, DIM), lambda t, te, ta: (t, 0)),
            pl.BlockSpec((B, 1), lambda t, te, ta: (t, 0)),
            pl.BlockSpec((1, DIM, HID), lambda t, te, ta: (te[t], 0, 0)),
            pl.BlockSpec((1, 1, HID), lambda t, te, ta: (te[t], 0, 0)),
            pl.BlockSpec((1, HID, DIM), lambda t, te, ta: (te[t], 0, 0)),
            pl.BlockSpec((1, 1, DIM), lambda t, te, ta: (te[t], 0, 0)),
        ],
        out_specs=pl.BlockSpec((B, DIM), lambda t, te, ta: (t, 0)),
    )
    return pl.pallas_call(
        _k4_body,
        grid_spec=grid_spec,
        out_shape=jax.ShapeDtypeStruct((PADTOT, DIM), jnp.float32),
        compiler_params=pltpu.CompilerParams(
            vmem_limit_bytes=120 * 1024 * 1024),
    )(tile_e, tile_act, xs, sw2, W1, b1r, W2, b2)


@functools.partial(
    pl.kernel,
    out_type=jax.ShapeDtypeStruct((N_TOKENS, DIM), jnp.float32),
    scratch_types=[
        [pltpu.VMEM((16,), jnp.int32) for _ in range(4)],
        [pltpu.VMEM((16,), jnp.int32) for _ in range(4)],
        [pltpu.VMEM((16, DIM), jnp.float32) for _ in range(2)],
        [pltpu.VMEM((16, DIM), jnp.float32) for _ in range(2)],
        pltpu.SemaphoreType.DMA,
        [pltpu.SemaphoreType.DMA for _ in range(2)],
    ],
    **_MESH,
)
def _k5(hs_hbm, inv_hbm, out_hbm, pa_vs, pb_vs, bufa, bufb, sem_g, sem_w):
    wid = _wid()
    base = wid * TOK_PER_SUB
    nch = TOK_PER_SUB // 16
    idx_loads = []
    for c in range(nch):
        tb = base + c * 16
        idx_loads.append(
            pltpu.async_copy(inv_hbm.at[pl.ds(tb, 16)], pa_vs[c], sem_g))
        idx_loads.append(
            pltpu.async_copy(inv_hbm.at[pl.ds(N_TOKENS + tb, 16)],
                             pb_vs[c], sem_g))
    for h in idx_loads:
        h.wait()

    def start_gathers(c):
        return (pltpu.async_copy(hs_hbm.at[pa_vs[c]], bufa[c % 2], sem_g),
                pltpu.async_copy(hs_hbm.at[pb_vs[c]], bufb[c % 2], sem_g))

    writes = [None] * nch
    pend = start_gathers(0)
    for c in range(nch):
        pend[0].wait()
        pend[1].wait()
        ba, bb = bufa[c % 2], bufb[c % 2]

        def col_body(j, _):
            for i in range(16):
                ba[i, pl.ds(j * 16, 16)] = (ba[i, pl.ds(j * 16, 16)]
                                            + bb[i, pl.ds(j * 16, 16)])
            return 0

        lax.fori_loop(0, DIM // 16, col_body, 0)
        writes[c] = pltpu.async_copy(
            ba, out_hbm.at[pl.ds(base + c * 16, 16), :], sem_w[c % 2])
        if c + 1 < nch:
            if c >= 1:
                writes[c - 1].wait()
            pend = start_gathers(c + 1)
    writes[nch - 2].wait()
    writes[nch - 1].wait()


def kernel(x, Wr, br, W1, b1, W2, b2):
    logits = _k1(x, Wr, br.reshape(1, N_EXPERTS))
    e1, e2, w1, w2, histb = _k2a(logits.reshape(N_TOKENS * N_EXPERTS))
    inv, sw, xs, tile_e, tile_act = _k2b(e1, e2, w1, w2, histb, x)
    hs = _k4(tile_e, tile_act, xs, sw.reshape(PADTOT, 1),
             W1.astype(jnp.bfloat16), b1.reshape(N_EXPERTS, 1, HID),
             W2.astype(jnp.bfloat16), b2.reshape(N_EXPERTS, 1, DIM))
    return _k5(hs, inv)


# speedup vs baseline: 1.1111x; 1.1111x over previous
import functools

import jax
import jax.numpy as jnp
from jax import lax
from jax.experimental import pallas as pl
from jax.experimental.pallas import tpu as pltpu
from jax.experimental.pallas import tpu_sc as plsc

DIM = 1024
N_EXPERTS = 8
N_TOKENS = 2048
HID = 4 * DIM
HBLK = 512
N_HBLK = HID // HBLK

B = 256
NPAIR = 2 * N_TOKENS
PADTOT = NPAIR + N_EXPERTS * B
NT = PADTOT // B

NSUB = 32
TOK_PER_SUB = N_TOKENS // NSUB
ROWS_PER_SUB = PADTOT // NSUB

_MESH = dict(mesh=plsc.VectorSubcoreMesh(core_axis_name="c",
                                         subcore_axis_name="s"),
             compiler_params=pltpu.CompilerParams(needs_layout_passes=False))


def _wid():
    return lax.axis_index("s") * 2 + lax.axis_index("c")


def _iota16():
    return lax.broadcasted_iota(jnp.int32, (16,), 0)


def _k1_body(x_ref, wr_ref, br_ref, out_ref):
    out_ref[...] = (jnp.dot(x_ref[...], wr_ref[...],
                            preferred_element_type=jnp.float32)
                    + br_ref[...])


def _k1(x, Wr, br2):
    return pl.pallas_call(
        _k1_body,
        in_specs=[
            pl.BlockSpec((N_TOKENS, DIM), lambda: (0, 0)),
            pl.BlockSpec((DIM, N_EXPERTS), lambda: (0, 0)),
            pl.BlockSpec((1, N_EXPERTS), lambda: (0, 0)),
        ],
        out_specs=pl.BlockSpec((N_TOKENS, N_EXPERTS), lambda: (0, 0)),
        out_shape=jax.ShapeDtypeStruct((N_TOKENS, N_EXPERTS), jnp.float32),
    )(x, Wr, br2)


@functools.partial(
    pl.kernel,
    out_type=(
        jax.ShapeDtypeStruct((N_TOKENS,), jnp.int32),
        jax.ShapeDtypeStruct((N_TOKENS,), jnp.int32),
        jax.ShapeDtypeStruct((N_TOKENS,), jnp.float32),
        jax.ShapeDtypeStruct((N_TOKENS,), jnp.float32),
        jax.ShapeDtypeStruct((NSUB * 8,), jnp.int32),
    ),
    scratch_types=[
        pltpu.VMEM((TOK_PER_SUB * N_EXPERTS,), jnp.float32),
        pltpu.VMEM((TOK_PER_SUB,), jnp.int32),
        pltpu.VMEM((TOK_PER_SUB,), jnp.int32),
        pltpu.VMEM((TOK_PER_SUB,), jnp.float32),
        pltpu.VMEM((TOK_PER_SUB,), jnp.float32),
        pltpu.VMEM((16,), jnp.int32),
    ],
    **_MESH,
)
def _k2a(logits_hbm, e1_hbm, e2_hbm, w1_hbm, w2_hbm, histb_hbm,
         lg_v, e1_v, e2_v, w1_v, w2_v, hist_v):
    wid = _wid()
    base = wid * TOK_PER_SUB
    pltpu.sync_copy(
        logits_hbm.at[pl.ds(base * N_EXPERTS, TOK_PER_SUB * N_EXPERTS)],
        lg_v)
    it = _iota16()
    idx_vecs = []
    for c in range(TOK_PER_SUB // 16):
        rows = (it + c * 16) * N_EXPERTS
        L = [plsc.load_gather(lg_v, [rows + e]) for e in range(N_EXPERTS)]
        m1 = L[0]
        for e in range(1, N_EXPERTS):
            m1 = jnp.maximum(m1, L[e])
        idx1 = jnp.full((16,), N_EXPERTS - 1, jnp.int32)
        for e in range(N_EXPERTS - 1, -1, -1):
            idx1 = jnp.where(L[e] == m1, e, idx1)
        L2 = [jnp.where(idx1 == e, -3.0e38, L[e]) for e in range(N_EXPERTS)]
        m2 = L2[0]
        for e in range(1, N_EXPERTS):
            m2 = jnp.maximum(m2, L2[e])
        idx2 = jnp.full((16,), N_EXPERTS - 1, jnp.int32)
        for e in range(N_EXPERTS - 1, -1, -1):
            idx2 = jnp.where(L2[e] == m2, e, idx2)
        t = jnp.exp(m2 - m1)
        s = 1.0 + t
        e1_v[pl.ds(c * 16, 16)] = idx1
        e2_v[pl.ds(c * 16, 16)] = idx2
        w1_v[pl.ds(c * 16, 16)] = 1.0 / s
        w2_v[pl.ds(c * 16, 16)] = t / s
        idx_vecs += [idx1, idx2]
    hvec = jnp.zeros((16,), jnp.int32)
    for e in range(N_EXPERTS):
        tot = jnp.int32(0)
        for v in idx_vecs:
            tot = tot + lax.reduce_sum_p.bind(
                (v == e).astype(jnp.int32), axes=(0,))
        hvec = jnp.where(it == e, tot, hvec)
    hist_v[...] = hvec
    pltpu.sync_copy(e1_v, e1_hbm.at[pl.ds(base, TOK_PER_SUB)])
    pltpu.sync_copy(e2_v, e2_hbm.at[pl.ds(base, TOK_PER_SUB)])
    pltpu.sync_copy(w1_v, w1_hbm.at[pl.ds(base, TOK_PER_SUB)])
    pltpu.sync_copy(w2_v, w2_hbm.at[pl.ds(base, TOK_PER_SUB)])
    pltpu.sync_copy(hist_v.at[pl.ds(0, 8)], histb_hbm.at[pl.ds(wid * 8, 8)])


@functools.partial(
    pl.kernel,
    out_type=(
        jax.ShapeDtypeStruct((NPAIR,), jnp.int32),
        jax.ShapeDtypeStruct((PADTOT,), jnp.float32),
        jax.ShapeDtypeStruct((PADTOT, DIM), jnp.float32),
        jax.ShapeDtypeStruct((32,), jnp.int32),
        jax.ShapeDtypeStruct((32,), jnp.int32),
    ),
    scratch_types=[
        pltpu.VMEM((NSUB * 8,), jnp.int32),
        pltpu.VMEM((TOK_PER_SUB,), jnp.int32),
        pltpu.VMEM((TOK_PER_SUB,), jnp.int32),
        pltpu.VMEM((TOK_PER_SUB,), jnp.float32),
        pltpu.VMEM((TOK_PER_SUB,), jnp.float32),
        pltpu.VMEM((TOK_PER_SUB, DIM), jnp.float32),
        pltpu.VMEM((TOK_PER_SUB,), jnp.int32),
        pltpu.VMEM((TOK_PER_SUB,), jnp.int32),
        pltpu.VMEM((128,), jnp.int32),
        pltpu.VMEM((128,), jnp.float32),
        pltpu.VMEM((16,), jnp.int32),
        pltpu.VMEM((16,), jnp.int32),
        pltpu.SemaphoreType.DMA,
        pltpu.SemaphoreType.DMA,
    ],
    **_MESH,
)
def _k2b(e1_hbm, e2_hbm, w1_hbm, w2_hbm, histb_hbm, x_hbm,
         inv_hbm, sw_hbm, xs_hbm, te_hbm, ta_hbm,
         histall_v, e1_v, e2_v, w1_v, w2_v, xrows_v, pos1_v, pos2_v,
         posall_v, wall_v, tstage_v, astage_v, sem_in, sem_out):
    wid = _wid()
    base = wid * TOK_PER_SUB
    loads = [
        pltpu.async_copy(x_hbm.at[pl.ds(base, TOK_PER_SUB), :], xrows_v,
                         sem_in),
        pltpu.async_copy(histb_hbm, histall_v, sem_in),
        pltpu.async_copy(e1_hbm.at[pl.ds(base, TOK_PER_SUB)], e1_v, sem_in),
        pltpu.async_copy(e2_hbm.at[pl.ds(base, TOK_PER_SUB)], e2_v, sem_in),
        pltpu.async_copy(w1_hbm.at[pl.ds(base, TOK_PER_SUB)], w1_v, sem_in),
        pltpu.async_copy(w2_hbm.at[pl.ds(base, TOK_PER_SUB)], w2_v, sem_in),
    ]
    for h in loads:
        h.wait()
    it = _iota16()

    def rsum(v):
        return lax.reduce_sum_p.bind(v, axes=(0,))

    tot = []
    pre = []
    for e in range(N_EXPERTS):
        clo = plsc.load_gather(histall_v, [it * 8 + e])
        chi = plsc.load_gather(histall_v, [(it + 16) * 8 + e])
        tot.append(rsum(clo) + rsum(chi))
        z = jnp.zeros((16,), jnp.int32)
        pre.append(rsum(jnp.where(it < wid, clo, z))
                   + rsum(jnp.where(it + 16 < wid, chi, z)))
    off = [jnp.int32(0)]
    for e in range(N_EXPERTS):
        off.append(off[e] + ((tot[e] + (B - 1)) // B) * B)
    start = [off[e] + pre[e] for e in range(N_EXPERTS)]

    run = {e: jnp.int32(0) for e in range(N_EXPERTS)}
    for slot, (vsrc, wsrc) in enumerate(((e1_v, w1_v), (e2_v, w2_v))):
        for c in range(TOK_PER_SUB // 16):
            v = vsrc[pl.ds(c * 16, 16)]
            pos = jnp.zeros((16,), jnp.int32)
            for e in range(N_EXPERTS):
                m = v == e
                mi = m.astype(jnp.int32)
                r = plsc.cumsum(mi) - 1
                pos = jnp.where(m, start[e] + run[e] + r, pos)
                run[e] = run[e] + rsum(mi)
            o = slot * TOK_PER_SUB + c * 16
            posall_v[pl.ds(o, 16)] = pos
            wall_v[pl.ds(o, 16)] = wsrc[pl.ds(c * 16, 16)]
            pdst = pos1_v if slot == 0 else pos2_v
            pdst[pl.ds(c * 16, 16)] = pos
    stores = [
        pltpu.async_copy(pos1_v, inv_hbm.at[pl.ds(base, TOK_PER_SUB)],
                         sem_out),
        pltpu.async_copy(pos2_v,
                         inv_hbm.at[pl.ds(N_TOKENS + base, TOK_PER_SUB)],
                         sem_out),
        pltpu.async_copy(wall_v, sw_hbm.at[posall_v], sem_out),
        pltpu.async_copy(xrows_v, xs_hbm.at[pos1_v], sem_out),
        pltpu.async_copy(xrows_v, xs_hbm.at[pos2_v], sem_out),
    ]
    for h in stores:
        h.wait()

    @pl.when(wid == 0)
    def _tiles():
        for chunk in range(2):
            t_ids = it + chunk * 16
            tB = t_ids * B
            act = (tB < off[N_EXPERTS]).astype(jnp.int32)
            te = jnp.zeros((16,), jnp.int32)
            for k in range(1, N_EXPERTS):
                te = te + (tB >= off[k]).astype(jnp.int32)
            tstage_v[...] = te
            astage_v[...] = act
            pltpu.sync_copy(tstage_v, te_hbm.at[pl.ds(chunk * 16, 16)])
            pltpu.sync_copy(astage_v, ta_hbm.at[pl.ds(chunk * 16, 16)])


def _k4_body(te_ref, ta_ref, xs_ref, sw_ref, w1_ref, b1_ref, w2_ref, b2_ref,
             out_ref):
    t = pl.program_id(0)

    @pl.when(ta_ref[t] == 1)
    def _active():
        x = xs_ref[...]
        acc = b2_ref[0] * 1.0
        for hb in range(N_HBLK):
            w1blk = w1_ref[0][:, hb * HBLK:(hb + 1) * HBLK]
            g = jnp.dot(x, w1blk.astype(jnp.float32),
                        preferred_element_type=jnp.float32)
            g = g + b1_ref[0][:, hb * HBLK:(hb + 1) * HBLK]
            g = g * 0.5 * (1.0 + lax.erf(g * 0.7071067811865476))
            w2blk = w2_ref[0][hb * HBLK:(hb + 1) * HBLK, :]
            acc = acc + jnp.dot(g, w2blk, preferred_element_type=jnp.float32)
        out_ref[...] = acc * sw_ref[...]


def _k4(tile_e, tile_act, xs, sw2, W1, b1r, W2, b2):
    grid_spec = pltpu.PrefetchScalarGridSpec(
        num_scalar_prefetch=2,
        grid=(NT,),
        in_specs=[
            pl.BlockSpec((B, DIM), lambda t, te, ta: (t, 0)),
            pl.BlockSpec((B, 1), lambda t, te, ta: (t, 0)),
            pl.BlockSpec((1, DIM, HID), lambda t, te, ta: (te[t], 0, 0)),
            pl.BlockSpec((1, 1, HID), lambda t, te, ta: (te[t], 0, 0)),
            pl.BlockSpec((1, HID, DIM), lambda t, te, ta: (te[t], 0, 0)),
            pl.BlockSpec((1, 1, DIM), lambda t, te, ta: (te[t], 0, 0)),
        ],
        out_specs=pl.BlockSpec((B, DIM), lambda t, te, ta: (t, 0)),
    )
    return pl.pallas_call(
        _k4_body,
        grid_spec=grid_spec,
        out_shape=jax.ShapeDtypeStruct((PADTOT, DIM), jnp.float32),
        compiler_params=pltpu.CompilerParams(
            vmem_limit_bytes=120 * 1024 * 1024),
    )(tile_e, tile_act, xs, sw2, W1, b1r, W2, b2)


@functools.partial(
    pl.kernel,
    out_type=jax.ShapeDtypeStruct((N_TOKENS, DIM), jnp.float32),
    scratch_types=[
        [pltpu.VMEM((16,), jnp.int32) for _ in range(4)],
        [pltpu.VMEM((16,), jnp.int32) for _ in range(4)],
        [pltpu.VMEM((16, DIM), jnp.float32) for _ in range(2)],
        [pltpu.VMEM((16, DIM), jnp.float32) for _ in range(2)],
        pltpu.SemaphoreType.DMA,
        [pltpu.SemaphoreType.DMA for _ in range(2)],
    ],
    **_MESH,
)
def _k5(hs_hbm, inv_hbm, out_hbm, pa_vs, pb_vs, bufa, bufb, sem_g, sem_w):
    wid = _wid()
    base = wid * TOK_PER_SUB
    nch = TOK_PER_SUB // 16
    idx_loads = []
    for c in range(nch):
        tb = base + c * 16
        idx_loads.append(
            pltpu.async_copy(inv_hbm.at[pl.ds(tb, 16)], pa_vs[c], sem_g))
        idx_loads.append(
            pltpu.async_copy(inv_hbm.at[pl.ds(N_TOKENS + tb, 16)],
                             pb_vs[c], sem_g))
    for h in idx_loads:
        h.wait()

    def start_gathers(c):
        return (pltpu.async_copy(hs_hbm.at[pa_vs[c]], bufa[c % 2], sem_g),
                pltpu.async_copy(hs_hbm.at[pb_vs[c]], bufb[c % 2], sem_g))

    writes = [None] * nch
    pend = start_gathers(0)
    for c in range(nch):
        pend[0].wait()
        pend[1].wait()
        ba, bb = bufa[c % 2], bufb[c % 2]

        def col_body(j, _):
            for i in range(16):
                ba[i, pl.ds(j * 16, 16)] = (ba[i, pl.ds(j * 16, 16)]
                                            + bb[i, pl.ds(j * 16, 16)])
            return 0

        lax.fori_loop(0, DIM // 16, col_body, 0)
        writes[c] = pltpu.async_copy(
            ba, out_hbm.at[pl.ds(base + c * 16, 16), :], sem_w[c % 2])
        if c + 1 < nch:
            if c >= 1:
                writes[c - 1].wait()
            pend = start_gathers(c + 1)
    writes[nch - 2].wait()
    writes[nch - 1].wait()


def kernel(x, Wr, br, W1, b1, W2, b2):
    logits = _k1(x, Wr, br.reshape(1, N_EXPERTS))
    e1, e2, w1, w2, histb = _k2a(logits.reshape(N_TOKENS * N_EXPERTS))
    inv, sw, xs, tile_e, tile_act = _k2b(e1, e2, w1, w2, histb, x)
    hs = _k4(tile_e, tile_act, xs, sw.reshape(PADTOT, 1),
             W1.astype(jnp.bfloat16), b1.reshape(N_EXPERTS, 1, HID),
             W2, b2.reshape(N_EXPERTS, 1, DIM))
    return _k5(hs, inv)


# speedup vs baseline: 1.1127x; 1.0014x over previous
import functools

import jax
import jax.numpy as jnp
from jax import lax
from jax.experimental import pallas as pl
from jax.experimental.pallas import tpu as pltpu
from jax.experimental.pallas import tpu_sc as plsc

DIM = 1024
N_EXPERTS = 8
N_TOKENS = 2048
HID = 4 * DIM
HBLK = 512
N_HBLK = HID // HBLK

B = 256
NPAIR = 2 * N_TOKENS
PADTOT = NPAIR + N_EXPERTS * B
NT = PADTOT // B

NSUB = 32
TOK_PER_SUB = N_TOKENS // NSUB

_MESH = dict(mesh=plsc.VectorSubcoreMesh(core_axis_name="c",
                                         subcore_axis_name="s"),
             compiler_params=pltpu.CompilerParams(needs_layout_passes=False))


def _wid():
    return lax.axis_index("s") * 2 + lax.axis_index("c")


def _iota16():
    return lax.broadcasted_iota(jnp.int32, (16,), 0)


def _k1_body(x_ref, wr_ref, br_ref, out_ref):
    out_ref[...] = (jnp.dot(x_ref[...], wr_ref[...],
                            preferred_element_type=jnp.float32)
                    + br_ref[...])


def _k1(x, Wr, br2):
    return pl.pallas_call(
        _k1_body,
        in_specs=[
            pl.BlockSpec((N_TOKENS, DIM), lambda: (0, 0)),
            pl.BlockSpec((DIM, N_EXPERTS), lambda: (0, 0)),
            pl.BlockSpec((1, N_EXPERTS), lambda: (0, 0)),
        ],
        out_specs=pl.BlockSpec((N_TOKENS, N_EXPERTS), lambda: (0, 0)),
        out_shape=jax.ShapeDtypeStruct((N_TOKENS, N_EXPERTS), jnp.float32),
    )(x, Wr, br2)


@functools.partial(
    pl.kernel,
    out_type=(
        jax.ShapeDtypeStruct((N_TOKENS,), jnp.int32),
        jax.ShapeDtypeStruct((N_TOKENS,), jnp.int32),
        jax.ShapeDtypeStruct((N_TOKENS,), jnp.float32),
        jax.ShapeDtypeStruct((N_TOKENS,), jnp.float32),
        jax.ShapeDtypeStruct((NSUB * 8,), jnp.int32),
    ),
    scratch_types=[
        pltpu.VMEM((TOK_PER_SUB * N_EXPERTS,), jnp.float32),
        pltpu.VMEM((TOK_PER_SUB,), jnp.int32),
        pltpu.VMEM((TOK_PER_SUB,), jnp.int32),
        pltpu.VMEM((TOK_PER_SUB,), jnp.float32),
        pltpu.VMEM((TOK_PER_SUB,), jnp.float32),
        pltpu.VMEM((16,), jnp.int32),
    ],
    **_MESH,
)
def _k2a(logits_hbm, e1_hbm, e2_hbm, w1_hbm, w2_hbm, histb_hbm,
         lg_v, e1_v, e2_v, w1_v, w2_v, hist_v):
    wid = _wid()
    base = wid * TOK_PER_SUB
    pltpu.sync_copy(
        logits_hbm.at[pl.ds(base * N_EXPERTS, TOK_PER_SUB * N_EXPERTS)],
        lg_v)
    it = _iota16()
    idx_vecs = []
    for c in range(TOK_PER_SUB // 16):
        rows = (it + c * 16) * N_EXPERTS
        L = [plsc.load_gather(lg_v, [rows + e]) for e in range(N_EXPERTS)]
        m1 = L[0]
        for e in range(1, N_EXPERTS):
            m1 = jnp.maximum(m1, L[e])
        idx1 = jnp.full((16,), N_EXPERTS - 1, jnp.int32)
        for e in range(N_EXPERTS - 1, -1, -1):
            idx1 = jnp.where(L[e] == m1, e, idx1)
        L2 = [jnp.where(idx1 == e, -3.0e38, L[e]) for e in range(N_EXPERTS)]
        m2 = L2[0]
        for e in range(1, N_EXPERTS):
            m2 = jnp.maximum(m2, L2[e])
        idx2 = jnp.full((16,), N_EXPERTS - 1, jnp.int32)
        for e in range(N_EXPERTS - 1, -1, -1):
            idx2 = jnp.where(L2[e] == m2, e, idx2)
        t = jnp.exp(m2 - m1)
        s = 1.0 + t
        e1_v[pl.ds(c * 16, 16)] = idx1
        e2_v[pl.ds(c * 16, 16)] = idx2
        w1_v[pl.ds(c * 16, 16)] = 1.0 / s
        w2_v[pl.ds(c * 16, 16)] = t / s
        idx_vecs += [idx1, idx2]
    hvec = jnp.zeros((16,), jnp.int32)
    for e in range(N_EXPERTS):
        tot = jnp.int32(0)
        for v in idx_vecs:
            tot = tot + lax.reduce_sum_p.bind(
                (v == e).astype(jnp.int32), axes=(0,))
        hvec = jnp.where(it == e, tot, hvec)
    hist_v[...] = hvec
    pltpu.sync_copy(e1_v, e1_hbm.at[pl.ds(base, TOK_PER_SUB)])
    pltpu.sync_copy(e2_v, e2_hbm.at[pl.ds(base, TOK_PER_SUB)])
    pltpu.sync_copy(w1_v, w1_hbm.at[pl.ds(base, TOK_PER_SUB)])
    pltpu.sync_copy(w2_v, w2_hbm.at[pl.ds(base, TOK_PER_SUB)])
    pltpu.sync_copy(hist_v.at[pl.ds(0, 8)], histb_hbm.at[pl.ds(wid * 8, 8)])


@functools.partial(
    pl.kernel,
    out_type=(
        jax.ShapeDtypeStruct((NPAIR,), jnp.int32),
        jax.ShapeDtypeStruct((PADTOT,), jnp.float32),
        jax.ShapeDtypeStruct((PADTOT, DIM), jnp.float32),
        jax.ShapeDtypeStruct((32,), jnp.int32),
        jax.ShapeDtypeStruct((32,), jnp.int32),
    ),
    scratch_types=[
        pltpu.VMEM((NSUB * 8,), jnp.int32),
        pltpu.VMEM((TOK_PER_SUB,), jnp.int32),
        pltpu.VMEM((TOK_PER_SUB,), jnp.int32),
        pltpu.VMEM((TOK_PER_SUB,), jnp.float32),
        pltpu.VMEM((TOK_PER_SUB,), jnp.float32),
        pltpu.VMEM((TOK_PER_SUB, DIM), jnp.float32),
        pltpu.VMEM((TOK_PER_SUB,), jnp.int32),
        pltpu.VMEM((TOK_PER_SUB,), jnp.int32),
        pltpu.VMEM((128,), jnp.int32),
        pltpu.VMEM((128,), jnp.float32),
        pltpu.VMEM((16,), jnp.int32),
        pltpu.VMEM((16,), jnp.int32),
        pltpu.SemaphoreType.DMA,
        pltpu.SemaphoreType.DMA,
    ],
    **_MESH,
)
def _k2b(e1_hbm, e2_hbm, w1_hbm, w2_hbm, histb_hbm, x_hbm,
         inv_hbm, sw_hbm, xs_hbm, te_hbm, ta_hbm,
         histall_v, e1_v, e2_v, w1_v, w2_v, xrows_v, pos1_v, pos2_v,
         posall_v, wall_v, tstage_v, astage_v, sem_in, sem_out):
    wid = _wid()
    base = wid * TOK_PER_SUB
    loads = [
        pltpu.async_copy(x_hbm.at[pl.ds(base, TOK_PER_SUB), :], xrows_v,
                         sem_in),
        pltpu.async_copy(histb_hbm, histall_v, sem_in),
        pltpu.async_copy(e1_hbm.at[pl.ds(base, TOK_PER_SUB)], e1_v, sem_in),
        pltpu.async_copy(e2_hbm.at[pl.ds(base, TOK_PER_SUB)], e2_v, sem_in),
        pltpu.async_copy(w1_hbm.at[pl.ds(base, TOK_PER_SUB)], w1_v, sem_in),
        pltpu.async_copy(w2_hbm.at[pl.ds(base, TOK_PER_SUB)], w2_v, sem_in),
    ]
    for h in loads:
        h.wait()
    it = _iota16()

    def rsum(v):
        return lax.reduce_sum_p.bind(v, axes=(0,))

    tot = []
    pre = []
    for e in range(N_EXPERTS):
        clo = plsc.load_gather(histall_v, [it * 8 + e])
        chi = plsc.load_gather(histall_v, [(it + 16) * 8 + e])
        tot.append(rsum(clo) + rsum(chi))
        z = jnp.zeros((16,), jnp.int32)
        pre.append(rsum(jnp.where(it < wid, clo, z))
                   + rsum(jnp.where(it + 16 < wid, chi, z)))
    off = [jnp.int32(0)]
    for e in range(N_EXPERTS):
        off.append(off[e] + ((tot[e] + (B - 1)) // B) * B)
    start = [off[e] + pre[e] for e in range(N_EXPERTS)]

    run = {e: jnp.int32(0) for e in range(N_EXPERTS)}
    for slot, (vsrc, wsrc) in enumerate(((e1_v, w1_v), (e2_v, w2_v))):
        for c in range(TOK_PER_SUB // 16):
            v = vsrc[pl.ds(c * 16, 16)]
            pos = jnp.zeros((16,), jnp.int32)
            for e in range(N_EXPERTS):
                m = v == e
                mi = m.astype(jnp.int32)
                r = plsc.cumsum(mi) - 1
                pos = jnp.where(m, start[e] + run[e] + r, pos)
                run[e] = run[e] + rsum(mi)
            o = slot * TOK_PER_SUB + c * 16
            posall_v[pl.ds(o, 16)] = pos
            wall_v[pl.ds(o, 16)] = wsrc[pl.ds(c * 16, 16)]
            pdst = pos1_v if slot == 0 else pos2_v
            pdst[pl.ds(c * 16, 16)] = pos
    stores = [
        pltpu.async_copy(pos1_v, inv_hbm.at[pl.ds(base, TOK_PER_SUB)],
                         sem_out),
        pltpu.async_copy(pos2_v,
                         inv_hbm.at[pl.ds(N_TOKENS + base, TOK_PER_SUB)],
                         sem_out),
        pltpu.async_copy(wall_v, sw_hbm.at[posall_v], sem_out),
        pltpu.async_copy(xrows_v, xs_hbm.at[pos1_v], sem_out),
        pltpu.async_copy(xrows_v, xs_hbm.at[pos2_v], sem_out),
    ]
    for h in stores:
        h.wait()

    @pl.when(wid == 0)
    def _tiles():
        for chunk in range(2):
            t_ids = it + chunk * 16
            tB = t_ids * B
            act = (tB < off[N_EXPERTS]).astype(jnp.int32)
            te = jnp.zeros((16,), jnp.int32)
            for k in range(1, N_EXPERTS):
                te = te + (tB >= off[k]).astype(jnp.int32)
            tstage_v[...] = te
            astage_v[...] = act
            pltpu.sync_copy(tstage_v, te_hbm.at[pl.ds(chunk * 16, 16)])
            pltpu.sync_copy(astage_v, ta_hbm.at[pl.ds(chunk * 16, 16)])


def _k4_body(te_ref, ta_ref, xs_ref, sw_ref, w1_ref, b1_ref, w2_ref, b2_ref,
             out_ref):
    t = pl.program_id(0)

    @pl.when(ta_ref[t] == 1)
    def _active():
        x = xs_ref[...]
        acc = b2_ref[0] * 1.0
        for hb in range(N_HBLK):
            w1blk = w1_ref[0][:, hb * HBLK:(hb + 1) * HBLK]
            g = jnp.dot(x, w1blk.astype(jnp.float32),
                        preferred_element_type=jnp.float32)
            g = g + b1_ref[0][:, hb * HBLK:(hb + 1) * HBLK]
            g = g * 0.5 * (1.0 + lax.erf(g * 0.7071067811865476))
            w2blk = w2_ref[0][hb * HBLK:(hb + 1) * HBLK, :]
            acc = acc + jnp.dot(g, w2blk, preferred_element_type=jnp.float32)
        out_ref[...] = acc * sw_ref[...]


def _k4(tile_e, tile_act, xs, sw2, W1, b1r, W2, b2):
    grid_spec = pltpu.PrefetchScalarGridSpec(
        num_scalar_prefetch=2,
        grid=(NT,),
        in_specs=[
            pl.BlockSpec((B, DIM), lambda t, te, ta: (t, 0)),
            pl.BlockSpec((B, 1), lambda t, te, ta: (t, 0)),
            pl.BlockSpec((1, DIM, HID), lambda t, te, ta: (te[t], 0, 0)),
            pl.BlockSpec((1, 1, HID), lambda t, te, ta: (te[t], 0, 0)),
            pl.BlockSpec((1, HID, DIM), lambda t, te, ta: (te[t], 0, 0)),
            pl.BlockSpec((1, 1, DIM), lambda t, te, ta: (te[t], 0, 0)),
        ],
        out_specs=pl.BlockSpec((B, DIM), lambda t, te, ta: (t, 0)),
    )
    return pl.pallas_call(
        _k4_body,
        grid_spec=grid_spec,
        out_shape=jax.ShapeDtypeStruct((PADTOT, DIM), jnp.float32),
        compiler_params=pltpu.CompilerParams(
            vmem_limit_bytes=120 * 1024 * 1024),
    )(tile_e, tile_act, xs, sw2, W1, b1r, W2, b2)


@functools.partial(
    pl.kernel,
    out_type=jax.ShapeDtypeStruct((N_TOKENS, DIM), jnp.float32),
    scratch_types=[
        [pltpu.VMEM((16,), jnp.int32) for _ in range(4)],
        [pltpu.VMEM((16,), jnp.int32) for _ in range(4)],
        [pltpu.VMEM((16, DIM), jnp.float32) for _ in range(2)],
        [pltpu.VMEM((16, DIM), jnp.float32) for _ in range(2)],
        pltpu.SemaphoreType.DMA,
        [pltpu.SemaphoreType.DMA for _ in range(2)],
    ],
    **_MESH,
)
def _k5(hs_hbm, inv_hbm, out_hbm, pa_vs, pb_vs, bufa, bufb, sem_g, sem_w):
    wid = _wid()
    base = wid * TOK_PER_SUB
    nch = TOK_PER_SUB // 16
    idx_loads = []
    for c in range(nch):
        tb = base + c * 16
        idx_loads.append(
            pltpu.async_copy(inv_hbm.at[pl.ds(tb, 16)], pa_vs[c], sem_g))
        idx_loads.append(
            pltpu.async_copy(inv_hbm.at[pl.ds(N_TOKENS + tb, 16)],
                             pb_vs[c], sem_g))
    for h in idx_loads:
        h.wait()

    def start_gathers(c):
        return (pltpu.async_copy(hs_hbm.at[pa_vs[c]], bufa[c % 2], sem_g),
                pltpu.async_copy(hs_hbm.at[pb_vs[c]], bufb[c % 2], sem_g))

    writes = [None] * nch
    pend = start_gathers(0)
    for c in range(nch):
        pend[0].wait()
        pend[1].wait()
        ba, bb = bufa[c % 2], bufb[c % 2]

        def col_body(j, _):
            for i in range(16):
                ba[i, pl.ds(j * 16, 16)] = (ba[i, pl.ds(j * 16, 16)]
                                            + bb[i, pl.ds(j * 16, 16)])
            return 0

        lax.fori_loop(0, DIM // 16, col_body, 0)
        writes[c] = pltpu.async_copy(
            ba, out_hbm.at[pl.ds(base + c * 16, 16), :], sem_w[c % 2])
        if c + 1 < nch:
            if c >= 1:
                writes[c - 1].wait()
            pend = start_gathers(c + 1)
    writes[nch - 2].wait()
    writes[nch - 1].wait()


def kernel(x, Wr, br, W1, b1, W2, b2):
    logits = _k1(x, Wr, br.reshape(1, N_EXPERTS))
    e1, e2, w1, w2, histb = _k2a(logits.reshape(N_TOKENS * N_EXPERTS))
    inv, sw, xs, tile_e, tile_act = _k2b(e1, e2, w1, w2, histb, x)
    hs = _k4(tile_e, tile_act, xs, sw.reshape(PADTOT, 1),
             W1.astype(jnp.bfloat16), b1.reshape(N_EXPERTS, 1, HID),
             W2, b2.reshape(N_EXPERTS, 1, DIM))
    return _k5(hs, inv)
